# baseline (speedup 1.0000x reference)
import functools

import jax
import jax.numpy as jnp
from jax import lax
from jax.experimental import pallas as pl
from jax.experimental.pallas import tpu as pltpu
from jax.experimental.pallas import tpu_sc as plsc

_S, _N = 60, 4096
_SN = _S * _N
_NARGS = 16
_D = 256
_AE = 64
_NCMD = 6
_TBL_ROWS = _NCMD * 256 + (_NARGS - 1) * 256

_NC, _NS = 2, 16
_NW = _NC * _NS
_P = 64
_NST = 4
_PER_W = _SN // _NW
_NBLK = _PER_W // _P
_NB_TOT = _SN // _P


def _build_table_body(at1_ref, w_ref, cmd_ref, b_ref, out_ref):
  at1 = at1_ref[...]
  t0 = jnp.dot(at1, w_ref[pl.ds(0, _AE), :],
               preferred_element_type=jnp.float32)
  t0 = t0 + b_ref[...]
  for c in range(_NCMD):
    out_ref[pl.ds(c * 256, 256), :] = t0 + cmd_ref[pl.ds(c, 1), :]
  for k in range(1, _NARGS):
    tk = jnp.dot(at1, w_ref[pl.ds(k * _AE, _AE), :],
                 preferred_element_type=jnp.float32)
    out_ref[pl.ds(_NCMD * 256 + (k - 1) * 256, 256), :] = tk


def _build_table(arg_table, W, cmd_table, b):
  at1 = arg_table[1:257]
  cmdp = jnp.pad(cmd_table, ((0, 2), (0, 0)))
  return pl.pallas_call(
      _build_table_body,
      out_shape=jax.ShapeDtypeStruct((_TBL_ROWS, _D), jnp.float32),
  )(at1, W, cmdp, b.reshape(1, _D))


def _accumulate(acc_v, st_v):
  def row(r, carry):
    for t in range(_D // 16):
      sl = pl.ds(t * 16, 16)
      plsc.addupdate(acc_v.at[r, sl], st_v[r, sl])
    return carry
  lax.fori_loop(0, _P, row, 0)


def _sc_body(slab_hbm, table_hbm, out_hbm, raw_v, idx_v, acc_v, *rest):
  sts = rest[:_NST]
  sem_a = rest[_NST]
  sems = rest[_NST + 1:]
  wid = lax.axis_index("s") * _NC + lax.axis_index("c")

  def block(j, carry):
    bidx = wid * _NBLK + j
    base = bidx * _P
    pltpu.sync_copy(slab_hbm.at[bidx], raw_v)
    for t in range(_P // 16):
      sl = pl.ds(t * 16, 16)
      idx_v[0, sl] = raw_v[0, sl] * 256 + raw_v[1, sl]
      for g in range(1, _NARGS):
        idx_v[g, sl] = raw_v[g + 1, sl] + (_NCMD * 256 + (g - 1) * 256)
    d_acc = pltpu.async_copy(table_hbm.at[idx_v.at[0]], acc_v, sem_a)
    descs = {}
    for g in range(1, min(_NST, _NARGS)):
      descs[g] = pltpu.async_copy(
          table_hbm.at[idx_v.at[g]], sts[g % _NST], sems[g % _NST])
    d_acc.wait()
    for g in range(1, _NARGS):
      nxt = g + _NST - 1
      if nxt < _NARGS:
        descs[nxt] = pltpu.async_copy(
            table_hbm.at[idx_v.at[nxt]], sts[nxt % _NST], sems[nxt % _NST])
      descs[g].wait()
      _accumulate(acc_v, sts[g % _NST])
    pltpu.sync_copy(acc_v, out_hbm.at[pl.ds(base, _P)])
    return carry

  lax.fori_loop(0, _NBLK, block, 0)


def _sc_gather_sum(slab, table):
  mesh = plsc.VectorSubcoreMesh(core_axis_name="c", subcore_axis_name="s")
  f = pl.kernel(
      _sc_body,
      out_type=jax.ShapeDtypeStruct((_SN, _D), jnp.float32),
      mesh=mesh,
      scratch_types=[
          pltpu.VMEM((_NARGS + 1, _P), jnp.int32),
          pltpu.VMEM((_NARGS, _P), jnp.int32),
          pltpu.VMEM((_P, _D), jnp.float32),
      ] + [pltpu.VMEM((_P, _D), jnp.float32) for _ in range(_NST)]
        + [pltpu.SemaphoreType.DMA for _ in range(_NST + 1)],
  )
  return f(slab, table)


def kernel(commands, args, cmd_table, arg_table, W, b):
  table = _build_table(arg_table, W, cmd_table, b)
  flat = jnp.concatenate(
      [commands.reshape(_SN, 1), args.reshape(_SN, _NARGS)], axis=1)
  slab = flat.reshape(_NB_TOT, _P, _NARGS + 1).swapaxes(1, 2)
  out = _sc_gather_sum(slab, table)
  return out.reshape(_S, _N, _D)


# speedup vs baseline: 1.6473x; 1.6473x over previous
import functools

import jax
import jax.numpy as jnp
from jax import lax
from jax.experimental import pallas as pl
from jax.experimental.pallas import tpu as pltpu
from jax.experimental.pallas import tpu_sc as plsc

_S, _N = 60, 4096
_SN = _S * _N
_NARGS = 16
_D = 256
_AE = 64
_NCMD = 6
_NPAIR = 7
_NG = 9
_A_ROWS = _NCMD * 256 + 256
_B_ROWS = _NPAIR * 65536

_NC, _NS = 2, 16
_NW = _NC * _NS
_P = 128
_NST = 2
_PER_W = _SN // _NW
_NBLK = _PER_W // _P
_NB_TOT = _SN // _P


def _stage1_body(at1_ref, w_ref, cmd_ref, b_ref, ta_ref, tmid_ref):
  at1 = at1_ref[...]
  t0 = jnp.dot(at1, w_ref[pl.ds(0, _AE), :],
               preferred_element_type=jnp.float32)
  t0 = t0 + b_ref[...]
  for c in range(_NCMD):
    ta_ref[pl.ds(c * 256, 256), :] = t0 + cmd_ref[pl.ds(c, 1), :]
  t15 = jnp.dot(at1, w_ref[pl.ds(15 * _AE, _AE), :],
                preferred_element_type=jnp.float32)
  ta_ref[pl.ds(_NCMD * 256, 256), :] = t15
  for k in range(1, 15):
    tk = jnp.dot(at1, w_ref[pl.ds(k * _AE, _AE), :],
                 preferred_element_type=jnp.float32)
    tmid_ref[k - 1] = tk


def _stage1(arg_table, W, cmd_table, b):
  at1 = arg_table[1:257]
  cmdp = jnp.pad(cmd_table, ((0, 2), (0, 0)))
  return pl.pallas_call(
      _stage1_body,
      out_shape=(jax.ShapeDtypeStruct((_A_ROWS, _D), jnp.float32),
                 jax.ShapeDtypeStruct((14, 256, _D), jnp.float32)),
  )(at1, W, cmdp, b.reshape(1, _D))


def _stage2_body(rowt_ref, colt_ref, out_ref):
  col = colt_ref[0]
  for a_loc in range(32):
    out_ref[pl.ds(a_loc * 256, 256), :] = col + rowt_ref[0, pl.ds(a_loc, 1), :]


def _stage2(tmid):
  return pl.pallas_call(
      _stage2_body,
      grid=(_NPAIR, 8),
      in_specs=[
          pl.BlockSpec((1, 32, _D), lambda j, i: (2 * j, i, 0)),
          pl.BlockSpec((1, 256, _D), lambda j, i: (2 * j + 1, 0, 0)),
      ],
      out_specs=pl.BlockSpec((32 * 256, _D), lambda j, i: (j * 8 + i, 0)),
      out_shape=jax.ShapeDtypeStruct((_B_ROWS, _D), jnp.float32),
  )(tmid, tmid)


def _accumulate(acc_v, st_v):
  def row(r, carry):
    for t in range(_D // 16):
      sl = pl.ds(t * 16, 16)
      plsc.addupdate(acc_v.at[r, sl], st_v[r, sl])
    return carry
  lax.fori_loop(0, _P, row, 0)


def _sc_body(slab_hbm, ta_hbm, tb_hbm, out_hbm, raw_v, idx_v, acc_v, *rest):
  sts = rest[:_NST]
  sem_a = rest[_NST]
  sems = rest[_NST + 1:]
  wid = lax.axis_index("s") * _NC + lax.axis_index("c")
  tabs = [ta_hbm] + [tb_hbm] * _NPAIR + [ta_hbm]

  def block(j, carry):
    bidx = wid * _NBLK + j
    base = bidx * _P
    pltpu.sync_copy(slab_hbm.at[bidx], raw_v)
    for t in range(_P // 16):
      sl = pl.ds(t * 16, 16)
      idx_v[0, sl] = raw_v[0, sl] * 256 + raw_v[1, sl]
      for g in range(1, 1 + _NPAIR):
        idx_v[g, sl] = (raw_v[2 * g, sl] * 256 + raw_v[2 * g + 1, sl]
                        + (g - 1) * 65536)
      idx_v[_NG - 1, sl] = raw_v[16, sl] + _NCMD * 256
    d_acc = pltpu.async_copy(ta_hbm.at[idx_v.at[0]], acc_v, sem_a)
    descs = {}
    for g in range(1, min(_NST, _NG)):
      descs[g] = pltpu.async_copy(
          tabs[g].at[idx_v.at[g]], sts[g % _NST], sems[g % _NST])
    d_acc.wait()
    for g in range(1, _NG):
      nxt = g + _NST - 1
      if nxt < _NG:
        descs[nxt] = pltpu.async_copy(
            tabs[nxt].at[idx_v.at[nxt]], sts[nxt % _NST], sems[nxt % _NST])
      descs[g].wait()
      _accumulate(acc_v, sts[g % _NST])
    pltpu.sync_copy(acc_v, out_hbm.at[pl.ds(base, _P)])
    return carry

  lax.fori_loop(0, _NBLK, block, 0)


def _sc_gather_sum(slab, table_a, table_b):
  mesh = plsc.VectorSubcoreMesh(core_axis_name="c", subcore_axis_name="s")
  f = pl.kernel(
      _sc_body,
      out_type=jax.ShapeDtypeStruct((_SN, _D), jnp.float32),
      mesh=mesh,
      scratch_types=[
          pltpu.VMEM((_NARGS + 1, _P), jnp.int32),
          pltpu.VMEM((_NG, _P), jnp.int32),
          pltpu.VMEM((_P, _D), jnp.float32),
      ] + [pltpu.VMEM((_P, _D), jnp.float32) for _ in range(_NST)]
        + [pltpu.SemaphoreType.DMA for _ in range(_NST + 1)],
  )
  return f(slab, table_a, table_b)


def kernel(commands, args, cmd_table, arg_table, W, b):
  table_a, tmid = _stage1(arg_table, W, cmd_table, b)
  table_b = _stage2(tmid)
  flat = jnp.concatenate(
      [commands.reshape(_SN, 1), args.reshape(_SN, _NARGS)], axis=1)
  slab = flat.reshape(_NB_TOT, _P, _NARGS + 1).swapaxes(1, 2)
  out = _sc_gather_sum(slab, table_a, table_b)
  return out.reshape(_S, _N, _D)
